# Initial kernel scaffold; baseline (speedup 1.0000x reference)
import functools

import jax
import jax.numpy as jnp
from jax import lax
from jax.experimental import pallas as pl
from jax.experimental.pallas import tpu as pltpu
from jax.experimental.pallas import tpu_sc as plsc

N_NODES = 10000
N_EDGES = 320000
D_IN = 128
D_HID = 128
N_CLASSES = 64

NPAD = 10240
NC = 2
NS = 16
NW = NC * NS
EPW = N_EDGES // NW
CH = 80
NCHUNK = EPW // CH
RPT = NPAD // NS

_MESH = plsc.VectorSubcoreMesh(
    core_axis_name="c", subcore_axis_name="s", num_cores=NC, num_subcores=NS
)

_F32 = jnp.float32
_ZV = jnp.zeros((16,), _F32)


def _deg_body(src_hbm, dst_hbm, out_hbm, idx_v, ones_v, zbuf, deg_s, deg_d):
    c = lax.axis_index("c")
    s = lax.axis_index("s")
    for j in range(CH // 16):
        ones_v[pl.ds(j * 16, 16)] = jnp.ones((16,), _F32)
    for j in range(RPT // 16):
        zbuf[pl.ds(j * 16, 16)] = _ZV
    zoff = pl.multiple_of(s * RPT, 8)
    pltpu.sync_copy(zbuf, deg_s.at[pl.ds(zoff, RPT)])
    pltpu.sync_copy(zbuf, deg_d.at[pl.ds(zoff, RPT)])
    plsc.subcore_barrier()
    ebase = (c * NS + s) * EPW

    def body(i, carry):
        st = pl.multiple_of(ebase + i * CH, 8)
        pltpu.sync_copy(src_hbm.at[pl.ds(st, CH)], idx_v)
        pltpu.sync_copy(ones_v, deg_s.at[idx_v], add=True)
        pltpu.sync_copy(dst_hbm.at[pl.ds(st, CH)], idx_v)
        pltpu.sync_copy(ones_v, deg_d.at[idx_v], add=True)
        return carry

    lax.fori_loop(0, NCHUNK, body, 0)
    plsc.subcore_barrier()
    pltpu.sync_copy(deg_s.at[pl.ds(zoff, RPT)], out_hbm.at[c, 0, pl.ds(zoff, RPT)])
    pltpu.sync_copy(deg_d.at[pl.ds(zoff, RPT)], out_hbm.at[c, 1, pl.ds(zoff, RPT)])


_deg_call = pl.kernel(
    _deg_body,
    out_type=jax.ShapeDtypeStruct((NC, 2, NPAD), _F32),
    mesh=_MESH,
    scratch_types=[
        pltpu.VMEM((CH,), jnp.int32),
        pltpu.VMEM((CH,), _F32),
        pltpu.VMEM((RPT,), _F32),
        pltpu.VMEM_SHARED((NPAD,), _F32),
        pltpu.VMEM_SHARED((NPAD,), _F32),
    ],
)


def _make_agg(d):
    def _agg_body(p_hbm, src_hbm, dst_hbm, out_hbm, idx_s, idx_d, rows, agg_sh):
        c = lax.axis_index("c")
        s = lax.axis_index("s")

        def zbody(r, carry):
            for k in range(d // 16):
                rows[r, pl.ds(k * 16, 16)] = _ZV
            return carry

        lax.fori_loop(0, CH, zbody, 0)
        for t in range(RPT // CH):
            roff = pl.multiple_of(s * RPT + t * CH, 8)
            pltpu.sync_copy(rows, agg_sh.at[pl.ds(roff, CH)])
        plsc.subcore_barrier()
        ebase = (c * NS + s) * EPW

        def body(i, carry):
            st = pl.multiple_of(ebase + i * CH, 8)
            pltpu.sync_copy(src_hbm.at[pl.ds(st, CH)], idx_s)
            pltpu.sync_copy(dst_hbm.at[pl.ds(st, CH)], idx_d)
            pltpu.sync_copy(p_hbm.at[idx_s], rows)
            pltpu.sync_copy(rows, agg_sh.at[idx_d], add=True)
            return carry

        lax.fori_loop(0, NCHUNK, body, 0)
        plsc.subcore_barrier()
        roff = pl.multiple_of(s * RPT, 8)
        pltpu.sync_copy(agg_sh.at[pl.ds(roff, RPT)], out_hbm.at[c, pl.ds(roff, RPT)])

    return pl.kernel(
        _agg_body,
        out_type=jax.ShapeDtypeStruct((NC, NPAD, d), _F32),
        mesh=_MESH,
        scratch_types=[
            pltpu.VMEM((CH,), jnp.int32),
            pltpu.VMEM((CH,), jnp.int32),
            pltpu.VMEM((CH, d), _F32),
            pltpu.VMEM_SHARED((NPAD, d), _F32),
        ],
    )


_agg128 = _make_agg(D_HID)
_agg64 = _make_agg(N_CLASSES)

BR = 1024


def _norm(d0, d1):
    deg = d0 + d1
    return lax.rsqrt(jnp.where(deg > 0.0, deg, 1.0))


def _scale_mm_body(x_ref, w_ref, d0_ref, d1_ref, o_ref):
    ns = _norm(d0_ref[...], d1_ref[...])
    o_ref[...] = jnp.dot(x_ref[...], w_ref[...], preferred_element_type=_F32) * ns


def _mid_body(p0_ref, p1_ref, dd0_ref, dd1_ref, b1_ref, w2_ref, ds0_ref, ds1_ref, o_ref):
    nd = _norm(dd0_ref[...], dd1_ref[...])
    h = jnp.maximum((p0_ref[...] + p1_ref[...]) * nd + b1_ref[...], 0.0)
    ns = _norm(ds0_ref[...], ds1_ref[...])
    o_ref[...] = jnp.dot(h, w2_ref[...], preferred_element_type=_F32) * ns


def _fin_body(p0_ref, p1_ref, dd0_ref, dd1_ref, b2_ref, o_ref):
    nd = _norm(dd0_ref[...], dd1_ref[...])
    o_ref[...] = (p0_ref[...] + p1_ref[...]) * nd + b2_ref[...]


def _col_spec():
    return pl.BlockSpec((BR, 1), lambda i: (i, 0))


def _row_spec(d):
    return pl.BlockSpec((BR, d), lambda i: (i, 0))


def _full_spec(a, b):
    return pl.BlockSpec((a, b), lambda i: (0, 0))


def _scale_mm(x, w, d0, d1):
    d = w.shape[1]
    return pl.pallas_call(
        _scale_mm_body,
        grid=(NPAD // BR,),
        in_specs=[_row_spec(x.shape[1]), _full_spec(*w.shape), _col_spec(), _col_spec()],
        out_specs=_row_spec(d),
        out_shape=jax.ShapeDtypeStruct((NPAD, d), _F32),
    )(x, w, d0, d1)


def _mid(p0, p1, dd0, dd1, b1, w2, ds0, ds1):
    return pl.pallas_call(
        _mid_body,
        grid=(NPAD // BR,),
        in_specs=[
            _row_spec(D_HID), _row_spec(D_HID), _col_spec(), _col_spec(),
            _full_spec(1, D_HID), _full_spec(D_HID, N_CLASSES), _col_spec(), _col_spec(),
        ],
        out_specs=_row_spec(N_CLASSES),
        out_shape=jax.ShapeDtypeStruct((NPAD, N_CLASSES), _F32),
    )(p0, p1, dd0, dd1, b1, w2, ds0, ds1)


def _fin(p0, p1, dd0, dd1, b2):
    return pl.pallas_call(
        _fin_body,
        grid=(NPAD // BR,),
        in_specs=[
            _row_spec(N_CLASSES), _row_spec(N_CLASSES), _col_spec(), _col_spec(),
            _full_spec(1, N_CLASSES),
        ],
        out_specs=_row_spec(N_CLASSES),
        out_shape=jax.ShapeDtypeStruct((NPAD, N_CLASSES), _F32),
    )(p0, p1, dd0, dd1, b2)


def kernel(inputs, edge_index, W1, b1, W2, b2):
    x = jnp.pad(inputs, ((0, NPAD - N_NODES), (0, 0)))
    src = edge_index[0]
    dst = edge_index[1]

    deg = _deg_call(src, dst)
    d_s = deg[:, 0, :].reshape(NC, NPAD, 1)
    d_d = deg[:, 1, :].reshape(NC, NPAD, 1)

    p1 = _scale_mm(x, W1, d_s[0], d_s[1])
    parts1 = _agg128(p1, src, dst)
    p2 = _mid(parts1[0], parts1[1], d_d[0], d_d[1],
              b1.reshape(1, D_HID), W2, d_s[0], d_s[1])
    parts2 = _agg64(p2, src, dst)
    out = _fin(parts2[0], parts2[1], d_d[0], d_d[1], b2.reshape(1, N_CLASSES))
    return out[:N_NODES]


# speedup vs baseline: 4.7686x; 4.7686x over previous
import functools

import jax
import jax.numpy as jnp
from jax import lax
from jax.experimental import pallas as pl
from jax.experimental.pallas import tpu as pltpu
from jax.experimental.pallas import tpu_sc as plsc

N_NODES = 10000
N_EDGES = 320000
D_IN = 128
D_HID = 128
N_CLASSES = 64

NPAD = 10240
NC = 2
NS = 16
NW = NC * NS
EPW = N_EDGES // NW
CH = 80
NCHUNK = EPW // CH
RPT = NPAD // NS

_MESH = plsc.VectorSubcoreMesh(
    core_axis_name="c", subcore_axis_name="s", num_cores=NC, num_subcores=NS
)

_F32 = jnp.float32


def _zv():
    return jnp.zeros((16,), _F32)


def _deg_body(src_hbm, dst_hbm, out_hbm, idx_v, ones_v, zbuf, deg_s, deg_d):
    c = lax.axis_index("c")
    s = lax.axis_index("s")
    for j in range(CH // 16):
        ones_v[pl.ds(j * 16, 16)] = jnp.ones((16,), _F32)
    for j in range(RPT // 16):
        zbuf[pl.ds(j * 16, 16)] = _zv()
    zoff = pl.multiple_of(s * RPT, 8)
    pltpu.sync_copy(zbuf, deg_s.at[pl.ds(zoff, RPT)])
    pltpu.sync_copy(zbuf, deg_d.at[pl.ds(zoff, RPT)])
    plsc.subcore_barrier()
    ebase = (c * NS + s) * EPW

    def body(i, carry):
        st = pl.multiple_of(ebase + i * CH, 8)
        pltpu.sync_copy(src_hbm.at[pl.ds(st, CH)], idx_v)
        pltpu.sync_copy(ones_v, deg_s.at[idx_v], add=True)
        pltpu.sync_copy(dst_hbm.at[pl.ds(st, CH)], idx_v)
        pltpu.sync_copy(ones_v, deg_d.at[idx_v], add=True)
        return carry

    lax.fori_loop(0, NCHUNK, body, 0)
    plsc.subcore_barrier()
    pltpu.sync_copy(deg_s.at[pl.ds(zoff, RPT)], out_hbm.at[c, 0, pl.ds(zoff, RPT)])
    pltpu.sync_copy(deg_d.at[pl.ds(zoff, RPT)], out_hbm.at[c, 1, pl.ds(zoff, RPT)])


_deg_call = pl.kernel(
    _deg_body,
    out_type=jax.ShapeDtypeStruct((NC, 2, NPAD), _F32),
    mesh=_MESH,
    scratch_types=[
        pltpu.VMEM((CH,), jnp.int32),
        pltpu.VMEM((CH,), _F32),
        pltpu.VMEM((RPT,), _F32),
        pltpu.VMEM_SHARED((NPAD,), _F32),
        pltpu.VMEM_SHARED((NPAD,), _F32),
    ],
)


def _make_agg(d):
    def _agg_body(p_hbm, src_hbm, dst_hbm, out_hbm, idx_s, idx_d, rows, agg_sh):
        c = lax.axis_index("c")
        s = lax.axis_index("s")

        def zbody(r, carry):
            for k in range(d // 16):
                rows[r, pl.ds(k * 16, 16)] = _zv()
            return carry

        lax.fori_loop(0, CH, zbody, 0)
        for t in range(RPT // CH):
            roff = pl.multiple_of(s * RPT + t * CH, 8)
            pltpu.sync_copy(rows, agg_sh.at[pl.ds(roff, CH)])
        plsc.subcore_barrier()
        ebase = (c * NS + s) * EPW

        def body(i, carry):
            st = pl.multiple_of(ebase + i * CH, 8)
            pltpu.sync_copy(src_hbm.at[pl.ds(st, CH)], idx_s)
            pltpu.sync_copy(dst_hbm.at[pl.ds(st, CH)], idx_d)
            pltpu.sync_copy(p_hbm.at[idx_s], rows)
            pltpu.sync_copy(rows, agg_sh.at[idx_d], add=True)
            return carry

        lax.fori_loop(0, NCHUNK, body, 0)
        plsc.subcore_barrier()
        roff = pl.multiple_of(s * RPT, 8)
        pltpu.sync_copy(agg_sh.at[pl.ds(roff, RPT)], out_hbm.at[c, pl.ds(roff, RPT)])

    return pl.kernel(
        _agg_body,
        out_type=jax.ShapeDtypeStruct((NC, NPAD, d), _F32),
        mesh=_MESH,
        scratch_types=[
            pltpu.VMEM((CH,), jnp.int32),
            pltpu.VMEM((CH,), jnp.int32),
            pltpu.VMEM((CH, d), _F32),
            pltpu.VMEM_SHARED((NPAD, d), _F32),
        ],
    )


_agg128 = _make_agg(D_HID)

BR = 1024


def _norm(d0, d1):
    deg = d0 + d1
    return lax.rsqrt(jnp.where(deg > 0.0, deg, 1.0))


def _scale_mm_body(x_ref, w_ref, d0_ref, d1_ref, o_ref):
    ns = _norm(d0_ref[...], d1_ref[...])
    o_ref[...] = jnp.dot(x_ref[...], w_ref[...], preferred_element_type=_F32) * ns


def _mid_body(p0_ref, p1_ref, dd0_ref, dd1_ref, b1_ref, w2_ref, ds0_ref, ds1_ref, o_ref):
    nd = _norm(dd0_ref[...], dd1_ref[...])
    h = jnp.maximum((p0_ref[...] + p1_ref[...]) * nd + b1_ref[...], 0.0)
    ns = _norm(ds0_ref[...], ds1_ref[...])
    o_ref[...] = jnp.dot(h, w2_ref[...], preferred_element_type=_F32) * ns


def _fin_body(p0_ref, p1_ref, dd0_ref, dd1_ref, b2_ref, o_ref):
    nd = _norm(dd0_ref[...], dd1_ref[...])
    agg = (p0_ref[...] + p1_ref[...])[:, :N_CLASSES]
    o_ref[...] = agg * nd + b2_ref[...]


def _col_spec():
    return pl.BlockSpec((BR, 1), lambda i: (i, 0))


def _row_spec(d):
    return pl.BlockSpec((BR, d), lambda i: (i, 0))


def _full_spec(a, b):
    return pl.BlockSpec((a, b), lambda i: (0, 0))


def _scale_mm(x, w, d0, d1):
    d = w.shape[1]
    return pl.pallas_call(
        _scale_mm_body,
        grid=(NPAD // BR,),
        in_specs=[_row_spec(x.shape[1]), _full_spec(*w.shape), _col_spec(), _col_spec()],
        out_specs=_row_spec(d),
        out_shape=jax.ShapeDtypeStruct((NPAD, d), _F32),
    )(x, w, d0, d1)


def _mid(p0, p1, dd0, dd1, b1, w2, ds0, ds1):
    return pl.pallas_call(
        _mid_body,
        grid=(NPAD // BR,),
        in_specs=[
            _row_spec(D_HID), _row_spec(D_HID), _col_spec(), _col_spec(),
            _full_spec(1, D_HID), _full_spec(D_HID, D_HID), _col_spec(), _col_spec(),
        ],
        out_specs=_row_spec(D_HID),
        out_shape=jax.ShapeDtypeStruct((NPAD, D_HID), _F32),
    )(p0, p1, dd0, dd1, b1, w2, ds0, ds1)


def _fin(p0, p1, dd0, dd1, b2):
    return pl.pallas_call(
        _fin_body,
        grid=(NPAD // BR,),
        in_specs=[
            _row_spec(D_HID), _row_spec(D_HID), _col_spec(), _col_spec(),
            _full_spec(1, N_CLASSES),
        ],
        out_specs=_row_spec(N_CLASSES),
        out_shape=jax.ShapeDtypeStruct((NPAD, N_CLASSES), _F32),
    )(p0, p1, dd0, dd1, b2)


def kernel(inputs, edge_index, W1, b1, W2, b2):
    x = jnp.pad(inputs, ((0, NPAD - N_NODES), (0, 0)))
    src = edge_index[0]
    dst = edge_index[1]

    deg = _deg_call(src, dst)
    d_s = deg[:, 0, :].reshape(NC, NPAD, 1)
    d_d = deg[:, 1, :].reshape(NC, NPAD, 1)

    p1 = _scale_mm(x, W1, d_s[0], d_s[1])
    parts1 = _agg128(p1, src, dst)
    w2p = jnp.pad(W2, ((0, 0), (0, D_HID - N_CLASSES)))
    p2 = _mid(parts1[0], parts1[1], d_d[0], d_d[1],
              b1.reshape(1, D_HID), w2p, d_s[0], d_s[1])
    parts2 = _agg128(p2, src, dst)
    out = _fin(parts2[0], parts2[1], d_d[0], d_d[1], b2.reshape(1, N_CLASSES))
    return out[:N_NODES]
